# baseline (speedup 1.0000x reference)
import functools

import jax
import jax.numpy as jnp
from jax import lax
from jax.experimental import pallas as pl
from jax.experimental.pallas import tpu as pltpu
from jax.experimental.pallas import tpu_sc as plsc

EMBED = 64
CHUNK = 128
NBUF = 4
LANES = 16


def _gather_call(idx_t, table, n_batch, n_cols):
    info = plsc.get_sparse_core_info()
    num_cores = info.num_cores
    n_workers = num_cores * info.num_subcores
    blocks_per_col = n_batch // CHUNK
    n_chunks = n_cols * blocks_per_col
    per_w = n_chunks // n_workers
    mesh = plsc.VectorSubcoreMesh(core_axis_name="c", subcore_axis_name="s")

    @functools.partial(
        pl.kernel,
        mesh=mesh,
        out_type=jax.ShapeDtypeStruct((n_cols, EMBED, n_batch), jnp.float32),
        compiler_params=pltpu.CompilerParams(
            use_tc_tiling_on_sc=False, needs_layout_passes=False
        ),
        scratch_types=[
            pltpu.VMEM((NBUF, CHUNK), jnp.int32),
            pltpu.VMEM((NBUF, CHUNK, EMBED), jnp.float32),
            pltpu.VMEM((EMBED, CHUNK), jnp.float32),
            pltpu.SemaphoreType.DMA,
        ],
    )
    def body(idx_hbm, table_hbm, out_hbm, idx_v, rows_v, trans_v, sem):
        wid = lax.axis_index("s") * num_cores + lax.axis_index("c")
        c0 = wid * per_w

        def stage_idx(c, b):
            j = c // blocks_per_col
            ib = c % blocks_per_col
            pltpu.sync_copy(
                idx_hbm.at[j, pl.ds(ib * CHUNK, CHUNK)], idx_v.at[b]
            )

        def start_gather(b):
            pltpu.async_copy(table_hbm.at[idx_v.at[b]], rows_v.at[b], sem)

        def wait_gather(b):
            pltpu.make_async_copy(
                table_hbm.at[idx_v.at[b]], rows_v.at[b], sem
            ).wait()

        def transpose_and_write(c, b):
            def col(cc, carry):
                ccv = jnp.zeros((LANES,), jnp.int32) + cc
                for r in range(CHUNK // LANES):
                    riv = jnp.arange(
                        r * LANES, (r + 1) * LANES, dtype=jnp.int32
                    )
                    trans_v[cc, pl.ds(r * LANES, LANES)] = plsc.load_gather(
                        rows_v.at[b], [riv, ccv]
                    )
                return carry

            lax.fori_loop(0, EMBED, col, 0, unroll=4)
            j = c // blocks_per_col
            ib = c % blocks_per_col
            pltpu.sync_copy(
                trans_v, out_hbm.at[j, :, pl.ds(ib * CHUNK, CHUNK)]
            )

        for b in range(NBUF):
            stage_idx(c0 + b, b)
            start_gather(b)

        def group(g, carry):
            for b in range(NBUF):
                c = c0 + g * NBUF + b
                wait_gather(b)
                transpose_and_write(c, b)
                stage_idx(c + NBUF, b)
                start_gather(b)
            return carry

        lax.fori_loop(0, per_w // NBUF - 1, group, 0)
        for b in range(NBUF):
            c = c0 + per_w - NBUF + b
            wait_gather(b)
            transpose_and_write(c, b)

    return body(idx_t, table)


def kernel(indices, table):
    n_batch, n_cols = indices.shape
    idx_t = indices.T.astype(jnp.int32)
    out_t = _gather_call(idx_t, table, n_batch, n_cols)
    return jnp.transpose(out_t, (2, 0, 1))


# speedup vs baseline: 1.7611x; 1.7611x over previous
import functools

import jax
import jax.numpy as jnp
from jax import lax
from jax.experimental import pallas as pl
from jax.experimental.pallas import tpu as pltpu
from jax.experimental.pallas import tpu_sc as plsc

CHUNK = 128
NBUF = 8


def _gather_call(idx3, table, n_workers, n_chunks, embed_dim):
    num_cores = plsc.get_sparse_core_info().num_cores
    b_per_w = n_chunks * CHUNK
    total = n_workers * b_per_w
    mesh = plsc.VectorSubcoreMesh(core_axis_name="c", subcore_axis_name="s")

    @functools.partial(
        pl.kernel,
        mesh=mesh,
        out_type=jax.ShapeDtypeStruct((total, embed_dim), jnp.float32),
        compiler_params=pltpu.CompilerParams(use_tc_tiling_on_sc=False),
        scratch_types=[
            pltpu.VMEM((n_chunks, CHUNK), jnp.int32),
            pltpu.VMEM((NBUF, CHUNK, embed_dim), jnp.float32),
            pltpu.SemaphoreType.DMA,
        ],
    )
    def body(idx_hbm, table_hbm, out_hbm, idx_v, rows_v, sem):
        wid = lax.axis_index("s") * num_cores + lax.axis_index("c")
        base = wid * b_per_w
        pltpu.sync_copy(idx_hbm.at[wid], idx_v)
        for b in range(NBUF):
            pltpu.async_copy(table_hbm.at[idx_v.at[b]], rows_v.at[b], sem)
        n_groups = n_chunks // NBUF

        def group(g, carry):
            for b in range(NBUF):
                j = g * NBUF + b
                pltpu.make_async_copy(
                    table_hbm.at[idx_v.at[j]], rows_v.at[b], sem
                ).wait()
                pltpu.sync_copy(
                    rows_v.at[b], out_hbm.at[pl.ds(base + j * CHUNK, CHUNK)]
                )
                pltpu.async_copy(
                    table_hbm.at[idx_v.at[j + NBUF]], rows_v.at[b], sem
                )
            return carry

        lax.fori_loop(0, n_groups - 1, group, 0)
        for b in range(NBUF):
            j = (n_groups - 1) * NBUF + b
            pltpu.make_async_copy(
                table_hbm.at[idx_v.at[j]], rows_v.at[b], sem
            ).wait()
            pltpu.sync_copy(
                rows_v.at[b], out_hbm.at[pl.ds(base + j * CHUNK, CHUNK)]
            )

    return body(idx3, table)


def kernel(indices, table):
    original_shape = indices.shape
    embed_dim = table.shape[1]
    flat = indices.reshape(-1).astype(jnp.int32)
    info = plsc.get_sparse_core_info()
    n_workers = info.num_cores * info.num_subcores
    n_chunks = flat.size // (n_workers * CHUNK)
    idx3 = flat.reshape(n_workers, n_chunks, CHUNK)
    out = _gather_call(idx3, table, n_workers, n_chunks, embed_dim)
    return out.reshape(original_shape + (embed_dim,))
